# baseline (speedup 1.0000x reference)
import functools

import jax
import jax.numpy as jnp
from jax import lax
from jax.experimental import pallas as pl
from jax.experimental.pallas import tpu as pltpu
from jax.experimental.pallas import tpu_sc as plsc

B = 16384
D = 16
NC = 2
NS = 16
NW = NC * NS
BPW = B // NW
CHUNK = 128
NCH = BPW // CHUNK


def _sc_gather(user_gmf, item_gmf, user_mlp, item_mlp, uidx, iidx):
  f32 = jnp.float32
  out = jax.ShapeDtypeStruct((B, D), f32)
  mesh = plsc.VectorSubcoreMesh(core_axis_name="c", subcore_axis_name="s")

  @functools.partial(
      pl.kernel,
      mesh=mesh,
      compiler_params=pltpu.CompilerParams(use_tc_tiling_on_sc=False),
      out_type=[out, out, out, out],
      scratch_types=[
          pltpu.VMEM((NCH, CHUNK), jnp.int32),
          pltpu.VMEM((NCH, CHUNK), jnp.int32),
          pltpu.VMEM((BPW, D), f32),
          pltpu.VMEM((BPW, D), f32),
          pltpu.VMEM((BPW, D), f32),
          pltpu.VMEM((BPW, D), f32),
          pltpu.SemaphoreType.DMA,
      ],
  )
  def k(ug_hbm, ig_hbm, um_hbm, im_hbm, ui_hbm, ii_hbm,
        oug, oig, oum, oim, ui_v, ii_v, ug_v, ig_v, um_v, im_v, sem):
    wid = lax.axis_index("s") * NC + lax.axis_index("c")
    base = wid * BPW
    for j in range(NCH):
      pltpu.sync_copy(ui_hbm.at[pl.ds(base + j * CHUNK, CHUNK)], ui_v.at[j])
      pltpu.sync_copy(ii_hbm.at[pl.ds(base + j * CHUNK, CHUNK)], ii_v.at[j])
    copies = []
    for j in range(NCH):
      sl = pl.ds(j * CHUNK, CHUNK)
      copies.append(pltpu.async_copy(ug_hbm.at[ui_v.at[j]], ug_v.at[sl], sem))
      copies.append(pltpu.async_copy(ig_hbm.at[ii_v.at[j]], ig_v.at[sl], sem))
      copies.append(pltpu.async_copy(um_hbm.at[ui_v.at[j]], um_v.at[sl], sem))
      copies.append(pltpu.async_copy(im_hbm.at[ii_v.at[j]], im_v.at[sl], sem))
    for c in copies:
      c.wait()
    pltpu.sync_copy(ug_v, oug.at[pl.ds(base, BPW)])
    pltpu.sync_copy(ig_v, oig.at[pl.ds(base, BPW)])
    pltpu.sync_copy(um_v, oum.at[pl.ds(base, BPW)])
    pltpu.sync_copy(im_v, oim.at[pl.ds(base, BPW)])

  return k(user_gmf, item_gmf, user_mlp, item_mlp, uidx, iidx)


BLK = 1024


def _tc_body(ug_r, ig_r, um_r, im_r, w0a_r, w0b_r, b0_r, w1_r, b1_r,
             w2_r, b2_r, wfa_r, wfb_r, bf_r, out_r):
  f32 = jnp.float32
  gmf = ug_r[...] * ig_r[...]
  h = jnp.dot(um_r[...], w0a_r[...], preferred_element_type=f32)
  h = h + jnp.dot(im_r[...], w0b_r[...], preferred_element_type=f32)
  h = jnp.maximum(h + b0_r[...], 0.0)
  h = jnp.maximum(jnp.dot(h, w1_r[...], preferred_element_type=f32) + b1_r[...], 0.0)
  h = jnp.maximum(jnp.dot(h, w2_r[...], preferred_element_type=f32) + b2_r[...], 0.0)
  out_r[...] = (jnp.dot(gmf, wfa_r[...], preferred_element_type=f32)
                + jnp.dot(h, wfb_r[...], preferred_element_type=f32)
                + bf_r[...])


def _tc_mlp(ug, ig, um, im, W0, b0, W1, b1, W2, b2, Wf, bf):
  f32 = jnp.float32
  grid = B // BLK
  row_spec = pl.BlockSpec((BLK, D), lambda i: (i, 0))

  def full(x):
    return pl.BlockSpec(x.shape, lambda i: tuple(0 for _ in x.shape))

  w0a, w0b = W0[:D], W0[D:]
  wfa, wfb = Wf[:D], Wf[D:]
  b0r = b0.reshape(1, -1)
  b1r = b1.reshape(1, -1)
  b2r = b2.reshape(1, -1)
  bfr = bf.reshape(1, -1)
  args = (ug, ig, um, im, w0a, w0b, b0r, W1, b1r, W2, b2r, wfa, wfb, bfr)
  in_specs = [row_spec, row_spec, row_spec, row_spec] + [full(a) for a in args[4:]]
  out = pl.pallas_call(
      _tc_body,
      grid=(grid,),
      in_specs=in_specs,
      out_specs=pl.BlockSpec((BLK, 1), lambda i: (i, 0)),
      out_shape=jax.ShapeDtypeStruct((B, 1), f32),
  )(*args)
  return out[:, 0]


def kernel(user_gmf, item_gmf, user_mlp, item_mlp, W0, b0, W1, b1, W2, b2,
           Wf, bf, user_indices, item_indices):
  uidx = user_indices.astype(jnp.int32)
  iidx = item_indices.astype(jnp.int32)
  ug, ig, um, im = _sc_gather(user_gmf, item_gmf, user_mlp, item_mlp, uidx, iidx)
  return _tc_mlp(ug, ig, um, im, W0, b0, W1, b1, W2, b2, Wf, bf)
